# baseline (speedup 1.0000x reference)
import jax
import jax.numpy as jnp
from jax.experimental import pallas as pl
from jax.experimental.pallas import tpu as pltpu

N = 4096
FEATS = 128
HID = 64
LABEL = 10
BI = 512
BK = 512
NI = N // BI
NK = N // BK


def _gcn_kernel(v_ref, adj_ref, w0_ref, wout_ref, bout_ref, out_ref,
                support_ref, acc_ref):
    i = pl.program_id(0)
    k = pl.program_id(1)

    @pl.when(i == 0)
    def _build_support():
        support_ref[pl.ds(k * BK, BK), :] = jnp.dot(
            v_ref[:], w0_ref[:], preferred_element_type=jnp.float32)

    @pl.when((i == 0) & (k == 0))
    def _init_out():
        out_ref[:] = bout_ref[:]

    part = jnp.dot(adj_ref[:], support_ref[pl.ds(k * BK, BK), :],
                   preferred_element_type=jnp.float32)

    @pl.when(k == 0)
    def _acc_init():
        acc_ref[:] = part

    @pl.when(k > 0)
    def _acc_add():
        acc_ref[:] += part

    @pl.when(k == NK - 1)
    def _finish_row():
        s = jnp.sum(jnp.maximum(acc_ref[:], 0.0), axis=1)[None, :]
        out_ref[:] += jax.lax.dot_general(
            s, wout_ref[:], (((1,), (1,)), ((), ())),
            preferred_element_type=jnp.float32)


def kernel(v, adj, W0, W_out, b_out):
    out = pl.pallas_call(
        _gcn_kernel,
        grid=(NI, NK),
        in_specs=[
            pl.BlockSpec((BK, FEATS), lambda i, k: (k, 0)),
            pl.BlockSpec((BI, BK), lambda i, k: (i, k)),
            pl.BlockSpec((FEATS, HID), lambda i, k: (0, 0)),
            pl.BlockSpec((LABEL, BI), lambda i, k: (0, i)),
            pl.BlockSpec((1, LABEL), lambda i, k: (0, 0)),
        ],
        out_specs=pl.BlockSpec((1, LABEL), lambda i, k: (0, 0)),
        out_shape=jax.ShapeDtypeStruct((1, LABEL), jnp.float32),
        scratch_shapes=[
            pltpu.VMEM((N, HID), jnp.float32),
            pltpu.VMEM((BI, HID), jnp.float32),
        ],
        compiler_params=pltpu.CompilerParams(
            dimension_semantics=("arbitrary", "arbitrary"),
        ),
    )(v, adj, W0, W_out, b_out.reshape(1, LABEL))
    return out.reshape(LABEL)


# speedup vs baseline: 2.3618x; 2.3618x over previous
import jax
import jax.numpy as jnp
from jax.experimental import pallas as pl
from jax.experimental.pallas import tpu as pltpu

N = 4096
FEATS = 128
HID = 64
LABEL = 10
BK = 512
NK = N // BK


def _gcn_kernel(v_ref, adj_ref, w0_ref, wout_ref, bout_ref, out_ref,
                acc_ref):
    k = pl.program_id(0)

    support_k = jnp.dot(v_ref[:], w0_ref[:],
                        preferred_element_type=jnp.float32)
    part = jnp.dot(adj_ref[:], support_k,
                   preferred_element_type=jnp.float32)

    @pl.when(k == 0)
    def _acc_init():
        acc_ref[:] = part

    @pl.when(k > 0)
    def _acc_add():
        acc_ref[:] += part

    @pl.when(k == NK - 1)
    def _finish():
        s = jnp.sum(jnp.maximum(acc_ref[:], 0.0), axis=1)[None, :]
        out_ref[:] = bout_ref[:] + jax.lax.dot_general(
            s, wout_ref[:], (((1,), (1,)), ((), ())),
            preferred_element_type=jnp.float32)


def kernel(v, adj, W0, W_out, b_out):
    out = pl.pallas_call(
        _gcn_kernel,
        grid=(NK,),
        in_specs=[
            pl.BlockSpec((BK, FEATS), lambda k: (k, 0)),
            pl.BlockSpec((N, BK), lambda k: (0, k)),
            pl.BlockSpec((FEATS, HID), lambda k: (0, 0)),
            pl.BlockSpec((LABEL, N), lambda k: (0, 0)),
            pl.BlockSpec((1, LABEL), lambda k: (0, 0)),
        ],
        out_specs=pl.BlockSpec((1, LABEL), lambda k: (0, 0)),
        out_shape=jax.ShapeDtypeStruct((1, LABEL), jnp.float32),
        scratch_shapes=[
            pltpu.VMEM((N, HID), jnp.float32),
        ],
    )(v, adj, W0, W_out, b_out.reshape(1, LABEL))
    return out.reshape(LABEL)
